# baseline (speedup 1.0000x reference)
import functools

import jax
import jax.numpy as jnp
from jax import lax
from jax.experimental import pallas as pl
from jax.experimental.pallas import tpu as pltpu
from jax.experimental.pallas import tpu_sc as plsc

_NC = 2
_NS = 16
_NW = _NC * _NS
_LANES = 16
_NB = 4


def _pos_embedding(emb_dim, seq_len):
    pos = jnp.arange(seq_len, dtype=jnp.float32)[:, None]
    i = jnp.arange(emb_dim, dtype=jnp.float32)[None, :]
    angle_rates = 1.0 / jnp.power(10000.0, (2.0 * jnp.floor(i / 2.0)) / float(emb_dim))
    angles = pos * angle_rates
    even = (jnp.arange(emb_dim)[None, :] % 2) == 0
    return jnp.where(even, jnp.sin(angles), jnp.cos(angles)).astype(jnp.float32)


def kernel(inputs, token_embeddings):
    batch, seq_len = inputs.shape
    _, emb_dim = token_embeddings.shape
    bpw = batch // _NW
    nblk = bpw // _LANES

    pe = _pos_embedding(emb_dim, seq_len)
    idx_t = inputs.T

    mesh = plsc.VectorSubcoreMesh(core_axis_name="c", subcore_axis_name="s")

    @functools.partial(
        pl.kernel,
        out_type=jax.ShapeDtypeStruct((seq_len, emb_dim, batch), jnp.float32),
        mesh=mesh,
        compiler_params=pltpu.CompilerParams(
            use_tc_tiling_on_sc=False, needs_layout_passes=False),
        scratch_types=[
            pltpu.VMEM((seq_len, bpw), jnp.int32),
            pltpu.VMEM((seq_len, emb_dim), jnp.float32),
            [pltpu.VMEM((bpw, emb_dim), jnp.float32) for _ in range(_NB)],
            [pltpu.VMEM((emb_dim, bpw), jnp.float32) for _ in range(_NB)],
            [pltpu.SemaphoreType.DMA for _ in range(_NB)],
            [pltpu.SemaphoreType.DMA for _ in range(_NB)],
        ],
    )
    def emb(idx_hbm, table_hbm, pe_hbm, out_hbm,
            idx_v, pe_v, gbufs, tbufs, gsems, ssems):
        wid = lax.axis_index("s") * _NC + lax.axis_index("c")
        b0 = wid * bpw
        pltpu.sync_copy(idx_hbm.at[:, pl.ds(b0, bpw)], idx_v)
        pltpu.sync_copy(pe_hbm, pe_v)

        def issue_gather(s, b):
            pltpu.async_copy(table_hbm.at[idx_v.at[s]], gbufs[b], gsems[b])

        def wait_gather(s, b):
            pltpu.make_async_copy(
                table_hbm.at[idx_v.at[s]], gbufs[b], gsems[b]).wait()

        def out_slice(s):
            return out_hbm.at[s, :, pl.ds(b0, bpw)]

        def issue_store(s, b):
            pltpu.async_copy(tbufs[b], out_slice(s), ssems[b])

        def wait_store(s, b):
            pltpu.make_async_copy(tbufs[b], out_slice(s), ssems[b]).wait()

        def transpose_add(s, b):
            gbuf = gbufs[b]
            tbuf = tbufs[b]

            svec = jnp.full((_LANES,), s, jnp.int32)

            def feat_body(d, carry):
                dvec = jnp.full((_LANES,), d, jnp.int32)
                pe_sd = plsc.load_gather(pe_v, [svec, dvec])
                for l in range(nblk):
                    rvec = lax.iota(jnp.int32, _LANES) + (l * _LANES)
                    v = plsc.load_gather(gbuf, [rvec, dvec])
                    tbuf[d, pl.ds(l * _LANES, _LANES)] = v + pe_sd
                return carry

            lax.fori_loop(0, emb_dim, feat_body, 0)

        for b in range(_NB):
            issue_gather(b, b)

        def body(s, carry):
            b = lax.rem(s, _NB)

            def stage(bb):
                wait_gather(s, bb)

                @pl.when(s >= _NB)
                def _():
                    wait_store(s - _NB, bb)

                transpose_add(s, bb)
                issue_store(s, bb)

                @pl.when(s + _NB < seq_len)
                def _():
                    issue_gather(s + _NB, bb)

            for bb in range(_NB):
                @pl.when(b == bb)
                def _(bb=bb):
                    stage(bb)
            return carry

        lax.fori_loop(0, seq_len, body, 0)

        for b in range(_NB):
            wait_store(seq_len - _NB + b, b)

    out_t = emb(idx_t, token_embeddings, pe)
    return jnp.transpose(out_t, (2, 0, 1))


# speedup vs baseline: 1.0890x; 1.0890x over previous
import functools

import jax
import jax.numpy as jnp
from jax import lax
from jax.experimental import pallas as pl
from jax.experimental.pallas import tpu as pltpu
from jax.experimental.pallas import tpu_sc as plsc

_NC = 2
_NS = 16
_NW = _NC * _NS
_LANES = 16
_TROW = 128
_NG = 2
_NT = 4


def _pos_embedding(emb_dim, seq_len):
    pos = jnp.arange(seq_len, dtype=jnp.float32)[:, None]
    i = jnp.arange(emb_dim, dtype=jnp.float32)[None, :]
    angle_rates = 1.0 / jnp.power(10000.0, (2.0 * jnp.floor(i / 2.0)) / float(emb_dim))
    angles = pos * angle_rates
    even = (jnp.arange(emb_dim)[None, :] % 2) == 0
    return jnp.where(even, jnp.sin(angles), jnp.cos(angles)).astype(jnp.float32)


def kernel(inputs, token_embeddings):
    batch, seq_len = inputs.shape
    vocab, emb_dim = token_embeddings.shape
    bpw = batch // _NW
    nblk = bpw // _LANES

    pe = _pos_embedding(emb_dim, seq_len)
    idx_t = inputs.T
    table2 = token_embeddings.reshape(vocab // 2, 2 * emb_dim)

    mesh = plsc.VectorSubcoreMesh(core_axis_name="c", subcore_axis_name="s")

    @functools.partial(
        pl.kernel,
        out_type=jax.ShapeDtypeStruct(
            (seq_len, emb_dim // 8, batch // _TROW, 8, _TROW), jnp.float32),
        mesh=mesh,
        compiler_params=pltpu.CompilerParams(
            use_tc_tiling_on_sc=False, needs_layout_passes=False),
        scratch_types=[
            pltpu.VMEM((seq_len, bpw), jnp.int32),
            pltpu.VMEM((seq_len, emb_dim), jnp.float32),
            [pltpu.VMEM((bpw,), jnp.int32) for _ in range(_NG)],
            [pltpu.VMEM((bpw, _TROW), jnp.float32) for _ in range(_NG)],
            [pltpu.VMEM((emb_dim // 8, 8, _TROW), jnp.float32)
             for _ in range(_NT)],
            [pltpu.SemaphoreType.DMA for _ in range(_NG)],
            [pltpu.SemaphoreType.DMA for _ in range(_NT)],
        ],
    )
    def emb(idx_hbm, table_hbm, pe_hbm, out_hbm,
            idx_v, pe_v, rbufs, gbufs, tbufs, gsems, ssems):
        wid = lax.axis_index("s") * _NC + lax.axis_index("c")
        b0 = wid * bpw
        pltpu.sync_copy(idx_hbm.at[:, pl.ds(b0, bpw)], idx_v)
        pltpu.sync_copy(pe_hbm, pe_v)

        rvecs = [lax.iota(jnp.int32, _LANES) + (l * _LANES)
                 for l in range(nblk)]

        def issue_gather(s, g):
            for l in range(nblk):
                sl = pl.ds(l * _LANES, _LANES)
                rbufs[g][sl] = jnp.right_shift(idx_v[s, sl], 1)
            pltpu.async_copy(table_hbm.at[rbufs[g]], gbufs[g], gsems[g])

        def wait_gather(s, g):
            pltpu.make_async_copy(
                table_hbm.at[rbufs[g]], gbufs[g], gsems[g]).wait()

        def out_slice(s):
            return out_hbm.at[s, :, wid]

        def issue_store(s, t):
            pltpu.async_copy(tbufs[t], out_slice(s), ssems[t])

        def wait_store(s, t):
            pltpu.make_async_copy(tbufs[t], out_slice(s), ssems[t]).wait()

        def transpose_add(s, g, t):
            gbuf = gbufs[g]
            tbuf = tbufs[t]
            svec = jnp.full((_LANES,), s, jnp.int32)
            hvecs = [
                jnp.left_shift(
                    jnp.bitwise_and(idx_v[s, pl.ds(l * _LANES, _LANES)], 1), 6)
                for l in range(nblk)
            ]

            def feat_body(d, carry):
                dvec = jnp.full((_LANES,), d, jnp.int32)
                pe_sd = plsc.load_gather(pe_v, [svec, dvec])
                dhi = jnp.right_shift(d, 3)
                dlo = jnp.bitwise_and(d, 7)
                for l in range(nblk):
                    v = plsc.load_gather(gbuf, [rvecs[l], hvecs[l] + dvec])
                    tbuf[dhi, dlo, pl.ds(l * _LANES, _LANES)] = v + pe_sd
                return carry

            lax.fori_loop(0, emb_dim, feat_body, 0, unroll=8)

        issue_gather(0, 0)
        issue_gather(1, 1)

        def body(jj, carry):
            for k in range(_NT):
                s = _NT * jj + k
                g = k % _NG
                t = k
                wait_gather(s, g)

                @pl.when(s >= _NT)
                def _():
                    wait_store(s - _NT, t)

                transpose_add(s, g, t)
                issue_store(s, t)

                @pl.when(s + _NG < seq_len)
                def _():
                    issue_gather(s + _NG, g)
            return carry

        lax.fori_loop(0, seq_len // _NT, body, 0)

        for t in range(_NT):
            wait_store(seq_len - _NT + t, t)

    out5 = emb(idx_t, table2, pe)
    out = jnp.transpose(out5, (2, 4, 0, 1, 3))
    return out.reshape(batch, seq_len, emb_dim)


# speedup vs baseline: 1.3309x; 1.2221x over previous
import functools

import jax
import jax.numpy as jnp
from jax import lax
from jax.experimental import pallas as pl
from jax.experimental.pallas import tpu as pltpu
from jax.experimental.pallas import tpu_sc as plsc

_NC = 2
_NS = 16
_NW = _NC * _NS
_LANES = 16
_TROW = 128
_NG = 2
_NT = 4


def _pos_embedding(emb_dim, seq_len):
    pos = jnp.arange(seq_len, dtype=jnp.float32)[:, None]
    i = jnp.arange(emb_dim, dtype=jnp.float32)[None, :]
    angle_rates = 1.0 / jnp.power(10000.0, (2.0 * jnp.floor(i / 2.0)) / float(emb_dim))
    angles = pos * angle_rates
    even = (jnp.arange(emb_dim)[None, :] % 2) == 0
    return jnp.where(even, jnp.sin(angles), jnp.cos(angles)).astype(jnp.float32)


def kernel(inputs, token_embeddings):
    batch, seq_len = inputs.shape
    vocab, emb_dim = token_embeddings.shape
    bpw = batch // _NW
    nblk = bpw // _LANES

    pe = _pos_embedding(emb_dim, seq_len)
    idx_t = inputs.T

    mesh = plsc.VectorSubcoreMesh(core_axis_name="c", subcore_axis_name="s")

    @functools.partial(
        pl.kernel,
        out_type=jax.ShapeDtypeStruct(
            (seq_len, emb_dim // 8, batch // _TROW, 8, _TROW), jnp.float32),
        mesh=mesh,
        compiler_params=pltpu.CompilerParams(
            use_tc_tiling_on_sc=False, needs_layout_passes=False),
        scratch_types=[
            pltpu.VMEM((seq_len, bpw), jnp.int32),
            pltpu.VMEM((seq_len, emb_dim), jnp.float32),
            [pltpu.VMEM((bpw, emb_dim), jnp.float32) for _ in range(_NG)],
            [pltpu.VMEM((emb_dim // 8, 8, _TROW), jnp.float32)
             for _ in range(_NT)],
            [pltpu.SemaphoreType.DMA for _ in range(_NG)],
            [pltpu.SemaphoreType.DMA for _ in range(_NT)],
        ],
    )
    def emb(idx_hbm, table_hbm, pe_hbm, out_hbm,
            idx_v, pe_v, gbufs, tbufs, gsems, ssems):
        wid = lax.axis_index("s") * _NC + lax.axis_index("c")
        b0 = wid * bpw
        pltpu.sync_copy(idx_hbm.at[:, pl.ds(b0, bpw)], idx_v)
        pltpu.sync_copy(pe_hbm, pe_v)

        rvecs = [lax.iota(jnp.int32, _LANES) + (l * _LANES)
                 for l in range(nblk)]

        def issue_gather(s, g):
            pltpu.async_copy(table_hbm.at[idx_v.at[s]], gbufs[g], gsems[g])

        def wait_gather(s, g):
            pltpu.make_async_copy(
                table_hbm.at[idx_v.at[s]], gbufs[g], gsems[g]).wait()

        def out_slice(s):
            return out_hbm.at[s, :, wid]

        def issue_store(s, t):
            pltpu.async_copy(tbufs[t], out_slice(s), ssems[t])

        def wait_store(s, t):
            pltpu.make_async_copy(tbufs[t], out_slice(s), ssems[t]).wait()

        def transpose_add(s, g, t):
            gbuf = gbufs[g]
            tbuf = tbufs[t]
            svec = jnp.full((_LANES,), s, jnp.int32)

            def feat_body(d, carry):
                dvec = jnp.full((_LANES,), d, jnp.int32)
                pe_sd = plsc.load_gather(pe_v, [svec, dvec])
                dhi = jnp.right_shift(d, 3)
                dlo = jnp.bitwise_and(d, 7)
                vs = [plsc.load_gather(gbuf, [rvecs[l], dvec])
                      for l in range(nblk)]
                for l in range(nblk):
                    tbuf[dhi, dlo, pl.ds(l * _LANES, _LANES)] = vs[l] + pe_sd
                return carry

            lax.fori_loop(0, emb_dim, feat_body, 0, unroll=4)

        issue_gather(0, 0)
        issue_gather(1, 1)

        def body(jj, carry):
            for k in range(_NT):
                s = _NT * jj + k
                g = k % _NG
                t = k
                wait_gather(s, g)

                @pl.when(s >= _NT)
                def _():
                    wait_store(s - _NT, t)

                transpose_add(s, g, t)
                issue_store(s, t)

                @pl.when(s + _NG < seq_len)
                def _():
                    issue_gather(s + _NG, g)
            return carry

        lax.fori_loop(0, seq_len // _NT, body, 0)

        for t in range(_NT):
            wait_store(seq_len - _NT + t, t)

    out5 = emb(idx_t, token_embeddings, pe)
    out = jnp.transpose(out5, (2, 4, 0, 1, 3))
    return out.reshape(batch, seq_len, emb_dim)


# speedup vs baseline: 1.3313x; 1.0003x over previous
import functools

import jax
import jax.numpy as jnp
from jax import lax
from jax.experimental import pallas as pl
from jax.experimental.pallas import tpu as pltpu
from jax.experimental.pallas import tpu_sc as plsc

_NC = 2
_NS = 16
_NW = _NC * _NS
_LANES = 16
_TROW = 128
_NG = 4
_NT = 4


def _pos_embedding(emb_dim, seq_len):
    pos = jnp.arange(seq_len, dtype=jnp.float32)[:, None]
    i = jnp.arange(emb_dim, dtype=jnp.float32)[None, :]
    angle_rates = 1.0 / jnp.power(10000.0, (2.0 * jnp.floor(i / 2.0)) / float(emb_dim))
    angles = pos * angle_rates
    even = (jnp.arange(emb_dim)[None, :] % 2) == 0
    return jnp.where(even, jnp.sin(angles), jnp.cos(angles)).astype(jnp.float32)


def kernel(inputs, token_embeddings):
    batch, seq_len = inputs.shape
    vocab, emb_dim = token_embeddings.shape
    bpw = batch // _NW
    nblk = bpw // _LANES

    pe = _pos_embedding(emb_dim, seq_len)
    idx_t = inputs.T

    mesh = plsc.VectorSubcoreMesh(core_axis_name="c", subcore_axis_name="s")

    @functools.partial(
        pl.kernel,
        out_type=jax.ShapeDtypeStruct(
            (seq_len, emb_dim // 8, batch // _TROW, 8, _TROW), jnp.float32),
        mesh=mesh,
        compiler_params=pltpu.CompilerParams(
            use_tc_tiling_on_sc=False, needs_layout_passes=False),
        scratch_types=[
            pltpu.VMEM((seq_len, bpw), jnp.int32),
            pltpu.VMEM((seq_len, emb_dim), jnp.float32),
            [pltpu.VMEM((bpw, emb_dim), jnp.float32) for _ in range(_NG)],
            [pltpu.VMEM((emb_dim // 8, 8, _TROW), jnp.float32)
             for _ in range(_NT)],
            [pltpu.SemaphoreType.DMA for _ in range(_NG)],
            [pltpu.SemaphoreType.DMA for _ in range(_NT)],
        ],
    )
    def emb(idx_hbm, table_hbm, pe_hbm, out_hbm,
            idx_v, pe_v, gbufs, tbufs, gsems, ssems):
        wid = lax.axis_index("s") * _NC + lax.axis_index("c")
        b0 = wid * bpw
        pltpu.sync_copy(idx_hbm.at[:, pl.ds(b0, bpw)], idx_v)
        pltpu.sync_copy(pe_hbm, pe_v)

        rvecs = [lax.iota(jnp.int32, _LANES) + (l * _LANES)
                 for l in range(nblk)]

        def issue_gather(s, g):
            pltpu.async_copy(table_hbm.at[idx_v.at[s]], gbufs[g], gsems[g])

        def wait_gather(s, g):
            pltpu.make_async_copy(
                table_hbm.at[idx_v.at[s]], gbufs[g], gsems[g]).wait()

        def out_slice(s):
            return out_hbm.at[s, :, wid]

        def issue_store(s, t):
            pltpu.async_copy(tbufs[t], out_slice(s), ssems[t])

        def wait_store(s, t):
            pltpu.make_async_copy(tbufs[t], out_slice(s), ssems[t]).wait()

        def transpose_add(s, g, t):
            gbuf = gbufs[g]
            tbuf = tbufs[t]
            svec = jnp.full((_LANES,), s, jnp.int32)

            def feat_body(d, carry):
                dvec = jnp.full((_LANES,), d, jnp.int32)
                pe_sd = plsc.load_gather(pe_v, [svec, dvec])
                dhi = jnp.right_shift(d, 3)
                dlo = jnp.bitwise_and(d, 7)
                vs = [plsc.load_gather(gbuf, [rvecs[l], dvec])
                      for l in range(nblk)]
                for l in range(nblk):
                    tbuf[dhi, dlo, pl.ds(l * _LANES, _LANES)] = vs[l] + pe_sd
                return carry

            lax.fori_loop(0, emb_dim, feat_body, 0, unroll=4)

        for g in range(_NG):
            issue_gather(g, g)

        def body(jj, carry):
            for k in range(_NT):
                s = _NT * jj + k
                g = k % _NG
                t = k
                wait_gather(s, g)

                @pl.when(s >= _NT)
                def _():
                    wait_store(s - _NT, t)

                transpose_add(s, g, t)
                issue_store(s, t)

                @pl.when(s + _NG < seq_len)
                def _():
                    issue_gather(s + _NG, g)
            return carry

        lax.fori_loop(0, seq_len // _NT, body, 0)

        for t in range(_NT):
            wait_store(seq_len - _NT + t, t)

    out5 = emb(idx_t, token_embeddings, pe)
    out = jnp.transpose(out5, (2, 4, 0, 1, 3))
    return out.reshape(batch, seq_len, emb_dim)
